# Initial kernel scaffold; baseline (speedup 1.0000x reference)
import jax
import jax.numpy as jnp
from jax.experimental import pallas as pl


def kernel(x, edge_index, params):
    raise NotImplementedError("write your pallas kernel here")


# speedup vs baseline: 4.7914x; 4.7914x over previous
import functools

import jax
import jax.numpy as jnp
from jax import lax
from jax.experimental import pallas as pl
from jax.experimental.pallas import tpu as pltpu
from jax.experimental.pallas import tpu_sc as plsc

N = 10000
E = 320000
D = 128
DO = 64
NLAYERS = 3

NC, NS, LANES = 2, 16, 16
NW = NC * NS
CHUNK = 128
CH = -(-E // (NW * CHUNK))
E_PAD = NW * CH * CHUNK
N_PAD = 10240
RZERO = N_PAD // NS

_sc_mesh = plsc.VectorSubcoreMesh(core_axis_name="c", subcore_axis_name="s")


@functools.partial(
    pl.kernel,
    out_type=jax.ShapeDtypeStruct((NC * N_PAD, D), jnp.float32),
    mesh=_sc_mesh,
    scratch_types=[
        pltpu.VMEM_SHARED((N_PAD, D), jnp.float32),
        pltpu.VMEM((CH, CHUNK), jnp.int32),
        pltpu.VMEM((CH, CHUNK), jnp.int32),
        pltpu.VMEM((CHUNK, D), jnp.float32),
        pltpu.SemaphoreType.DMA,
    ],
)
def _sc_aggregate(h_hbm, src_hbm, dst_hbm, out_hbm, acc_sh, src_v, dst_v, rows_v, sem):
    c = lax.axis_index("c")
    s = lax.axis_index("s")
    wid = s * NC + c

    def _zrow(i, carry):
        for j in range(D // LANES):
            rows_v[i, pl.ds(j * LANES, LANES)] = jnp.zeros((LANES,), jnp.float32)
        return carry

    lax.fori_loop(0, CHUNK, _zrow, 0)
    for k in range(RZERO // CHUNK):
        pltpu.sync_copy(rows_v, acc_sh.at[pl.ds(s * RZERO + k * CHUNK, CHUNK)])
    plsc.subcore_barrier()

    pltpu.sync_copy(src_hbm.at[wid], src_v)
    pltpu.sync_copy(dst_hbm.at[wid], dst_v)

    def _body(j, carry):
        pltpu.async_copy(h_hbm.at[src_v.at[j]], rows_v, sem).wait()
        pltpu.sync_copy(rows_v, acc_sh.at[dst_v.at[j]], add=True)
        return carry

    lax.fori_loop(0, CH, _body, 0)
    plsc.subcore_barrier()

    pltpu.sync_copy(
        acc_sh.at[pl.ds(s * RZERO, RZERO)],
        out_hbm.at[pl.ds(c * N_PAD + s * RZERO, RZERO)],
    )


def _tc_in_body(x_ref, w_ref, b_ref, o_ref):
    o_ref[...] = (
        jnp.dot(x_ref[...], w_ref[...], preferred_element_type=jnp.float32)
        + b_ref[0]
    )


_tc_in = pl.pallas_call(
    _tc_in_body, out_shape=jax.ShapeDtypeStruct((N, D), jnp.float32)
)


def _tc_mlp_body(h_ref, agg_ref, eps_ref, w1_ref, b1_ref, w2_ref, b2_ref,
                 g_ref, bb_ref, o_ref):
    z = (1.0 + eps_ref[0, 0]) * h_ref[...] + agg_ref[:N] + agg_ref[N_PAD:N_PAD + N]
    z = jnp.dot(z, w1_ref[...], preferred_element_type=jnp.float32) + b1_ref[0]
    z = jnp.maximum(z, 0.0)
    z = jnp.dot(z, w2_ref[...], preferred_element_type=jnp.float32) + b2_ref[0]
    m = jnp.mean(z, axis=0, keepdims=True)
    v = jnp.mean((z - m) ** 2, axis=0, keepdims=True)
    z = (z - m) * lax.rsqrt(v + 1e-5) * g_ref[0] + bb_ref[0]
    o_ref[...] = jnp.maximum(z, 0.0)


_tc_mlp = pl.pallas_call(
    _tc_mlp_body, out_shape=jax.ShapeDtypeStruct((N, D), jnp.float32)
)


def _tc_head_body(h0_ref, h1_ref, h2_ref, h3_ref, w1_ref, b1_ref, w2_ref,
                  b2_ref, o_ref):
    acc = jnp.dot(h0_ref[...], w1_ref[0:D], preferred_element_type=jnp.float32)
    acc += jnp.dot(h1_ref[...], w1_ref[D:2 * D], preferred_element_type=jnp.float32)
    acc += jnp.dot(h2_ref[...], w1_ref[2 * D:3 * D], preferred_element_type=jnp.float32)
    acc += jnp.dot(h3_ref[...], w1_ref[3 * D:4 * D], preferred_element_type=jnp.float32)
    c = jnp.maximum(acc + b1_ref[0], 0.0)
    o_ref[...] = jnp.dot(c, w2_ref[...], preferred_element_type=jnp.float32) + b2_ref[0]


_tc_head = pl.pallas_call(
    _tc_head_body, out_shape=jax.ShapeDtypeStruct((N, DO), jnp.float32)
)


def kernel(x, edge_index, params):
    src = edge_index[0].astype(jnp.int32)
    dst = edge_index[1].astype(jnp.int32)
    pad = E_PAD - E
    src_p = jnp.concatenate([src, jnp.zeros((pad,), jnp.int32)]).reshape(NW, CH, CHUNK)
    dst_p = jnp.concatenate([dst, jnp.full((pad,), N, jnp.int32)]).reshape(NW, CH, CHUNK)

    h = _tc_in(x, params["W_in"], params["b_in"].reshape(1, D))
    outs = [h]
    for i in range(NLAYERS):
        agg = _sc_aggregate(h, src_p, dst_p)
        h = _tc_mlp(
            h, agg,
            params[f"eps_{i}"].reshape(1, 1),
            params[f"W1_{i}"], params[f"b1_{i}"].reshape(1, D),
            params[f"W2_{i}"], params[f"b2_{i}"].reshape(1, D),
            params[f"bn_gamma_{i}"].reshape(1, D),
            params[f"bn_beta_{i}"].reshape(1, D),
        )
        outs.append(h)
    return _tc_head(
        outs[0], outs[1], outs[2], outs[3],
        params["W_c1"], params["b_c1"].reshape(1, D),
        params["W_c2"], params["b_c2"].reshape(1, DO),
    )
